# Initial kernel scaffold; baseline (speedup 1.0000x reference)
import jax
import jax.numpy as jnp
from jax.experimental import pallas as pl


def kernel(sites, bonds, sites_p, bonds_sp, bonds_ps, idx1, idx2, idx1_sp, idx2_sp, idx1_ps, idx2_ps, W_msg, b_msg, aW_msg, ab_msg, W_sp, b_sp, aW_sp, ab_sp, W_ps, b_ps, aW_ps, ab_ps, Wn1, bn1, Wn2, bn2, Wp1, bp1, Wp2, bp2):
    raise NotImplementedError("write your pallas kernel here")


# speedup vs baseline: 4.4533x; 4.4533x over previous
import functools

import jax
import jax.numpy as jnp
from jax import lax
from jax.experimental import pallas as pl

B = 16
N = 48
NP = 12
E = 768
ESP = 384
EPS = 384
IN = 16
MSG = 16
BOND = 12
HID = 32
OUT = 16


def _leaky(x):
    return jnp.maximum(x, 0.01 * x)


def _one_hot(idx_col, k):
    cols = lax.broadcasted_iota(jnp.int32, (idx_col.shape[0], k), 1)
    return (idx_col == cols).astype(jnp.float32)


def _message_block(s_src, s_rcv, bonds, idx1_col, idx2_col, k,
                   Wt, b_row, aWt, ab):
    oh1 = _one_hot(idx1_col, s_src.shape[0])
    oh2 = _one_hot(idx2_col, k)
    ss = jnp.dot(oh1, s_src, preferred_element_type=jnp.float32)
    sr = jnp.dot(oh2, s_rcv, preferred_element_type=jnp.float32)
    lat = (jnp.dot(ss, Wt[:IN], preferred_element_type=jnp.float32)
           + jnp.dot(sr, Wt[IN:2 * IN], preferred_element_type=jnp.float32)
           + jnp.dot(bonds, Wt[2 * IN:], preferred_element_type=jnp.float32)
           + b_row)
    lat = _leaky(lat)
    att = jax.nn.sigmoid(
        jnp.dot(lat, aWt, preferred_element_type=jnp.float32) + ab)
    lat = att * lat
    return lax.dot_general(oh2, lat, (((0,), (0,)), ((), ())),
                           preferred_element_type=jnp.float32)


def _body(sites_ref, bonds_ref, sites_p_ref, bonds_sp_ref, bonds_ps_ref,
          idx1_ref, idx2_ref, idx1_sp_ref, idx2_sp_ref,
          idx1_ps_ref, idx2_ps_ref,
          Wt_msg_ref, b_msg_ref, aWt_msg_ref, ab_msg_ref,
          Wt_sp_ref, b_sp_ref, aWt_sp_ref, ab_sp_ref,
          Wt_ps_ref, b_ps_ref, aWt_ps_ref, ab_ps_ref,
          Wn1t_ref, bn1_ref, Wn2t_ref, bn2_ref,
          Wp1t_ref, bp1_ref, Wp2t_ref, bp2_ref,
          sites_out_ref, sites_p_out_ref):
    s = sites_ref[0]
    sp = sites_p_ref[0]

    msg = _message_block(
        s, s, bonds_ref[0], idx1_ref[...], idx2_ref[...], N,
        Wt_msg_ref[...], b_msg_ref[...], aWt_msg_ref[...],
        ab_msg_ref[0, 0])
    msg_ps = _message_block(
        sp, s, bonds_ps_ref[0], idx1_ps_ref[...], idx2_ps_ref[...], N,
        Wt_ps_ref[...], b_ps_ref[...], aWt_ps_ref[...],
        ab_ps_ref[0, 0])
    msg_sp = _message_block(
        s, sp, bonds_sp_ref[0], idx1_sp_ref[...], idx2_sp_ref[...], NP,
        Wt_sp_ref[...], b_sp_ref[...], aWt_sp_ref[...],
        ab_sp_ref[0, 0])

    Wn1t = Wn1t_ref[...]
    h = _leaky(jnp.dot(s, Wn1t[:IN], preferred_element_type=jnp.float32)
               + jnp.dot(msg, Wn1t[IN:IN + MSG],
                         preferred_element_type=jnp.float32)
               + jnp.dot(msg_ps, Wn1t[IN + MSG:],
                         preferred_element_type=jnp.float32)
               + bn1_ref[...])
    sites_out_ref[0] = s + _leaky(
        jnp.dot(h, Wn2t_ref[...], preferred_element_type=jnp.float32)
        + bn2_ref[...])

    Wp1t = Wp1t_ref[...]
    hp = _leaky(jnp.dot(sp, Wp1t[:IN], preferred_element_type=jnp.float32)
                + jnp.dot(msg_sp, Wp1t[IN:],
                          preferred_element_type=jnp.float32)
                + bp1_ref[...])
    sites_p_out_ref[0] = sp + _leaky(
        jnp.dot(hp, Wp2t_ref[...], preferred_element_type=jnp.float32)
        + bp2_ref[...])


def _batch_spec(shape):
    return pl.BlockSpec((1,) + shape, lambda b: (b, 0, 0))


def _const_spec(shape):
    return pl.BlockSpec(shape, lambda b: tuple(0 for _ in shape))


@functools.partial(jax.jit, static_argnames=())
def _run(sites, bonds, sites_p, bonds_sp, bonds_ps,
         idx1c, idx2c, idx1_spc, idx2_spc, idx1_psc, idx2_psc,
         Wt_msg, b_msg, aWt_msg, ab_msg,
         Wt_sp, b_sp, aWt_sp, ab_sp,
         Wt_ps, b_ps, aWt_ps, ab_ps,
         Wn1t, bn1, Wn2t, bn2, Wp1t, bp1, Wp2t, bp2):
    grid = (B,)
    in_specs = [
        _batch_spec((N, IN)), _batch_spec((E, BOND)),
        _batch_spec((NP, IN)), _batch_spec((ESP, BOND)),
        _batch_spec((EPS, BOND)),
        _const_spec((E, 1)), _const_spec((E, 1)),
        _const_spec((ESP, 1)), _const_spec((ESP, 1)),
        _const_spec((EPS, 1)), _const_spec((EPS, 1)),
        _const_spec((2 * IN + BOND, MSG)), _const_spec((1, MSG)),
        _const_spec((MSG, 1)), _const_spec((1, 1)),
        _const_spec((2 * IN + BOND, MSG)), _const_spec((1, MSG)),
        _const_spec((MSG, 1)), _const_spec((1, 1)),
        _const_spec((2 * IN + BOND, MSG)), _const_spec((1, MSG)),
        _const_spec((MSG, 1)), _const_spec((1, 1)),
        _const_spec((IN + 2 * MSG, HID)), _const_spec((1, HID)),
        _const_spec((HID, OUT)), _const_spec((1, OUT)),
        _const_spec((IN + MSG, HID)), _const_spec((1, HID)),
        _const_spec((HID, OUT)), _const_spec((1, OUT)),
    ]
    out_specs = [_batch_spec((N, IN)), _batch_spec((NP, IN))]
    out_shapes = [
        jax.ShapeDtypeStruct((B, N, OUT), jnp.float32),
        jax.ShapeDtypeStruct((B, NP, OUT), jnp.float32),
    ]
    return pl.pallas_call(
        _body,
        grid=grid,
        in_specs=in_specs,
        out_specs=out_specs,
        out_shape=out_shapes,
    )(sites, bonds, sites_p, bonds_sp, bonds_ps,
      idx1c, idx2c, idx1_spc, idx2_spc, idx1_psc, idx2_psc,
      Wt_msg, b_msg, aWt_msg, ab_msg,
      Wt_sp, b_sp, aWt_sp, ab_sp,
      Wt_ps, b_ps, aWt_ps, ab_ps,
      Wn1t, bn1, Wn2t, bn2, Wp1t, bp1, Wp2t, bp2)


def kernel(sites, bonds, sites_p, bonds_sp, bonds_ps,
           idx1, idx2, idx1_sp, idx2_sp, idx1_ps, idx2_ps,
           W_msg, b_msg, aW_msg, ab_msg,
           W_sp, b_sp, aW_sp, ab_sp,
           W_ps, b_ps, aW_ps, ab_ps,
           Wn1, bn1, Wn2, bn2, Wp1, bp1, Wp2, bp2):
    col = lambda i: i.astype(jnp.int32).reshape(-1, 1)
    row = lambda v: v.reshape(1, -1)
    sites_new, sites_p_new = _run(
        sites, bonds, sites_p, bonds_sp, bonds_ps,
        col(idx1), col(idx2), col(idx1_sp), col(idx2_sp),
        col(idx1_ps), col(idx2_ps),
        W_msg.T, row(b_msg), aW_msg.T, ab_msg.reshape(1, 1),
        W_sp.T, row(b_sp), aW_sp.T, ab_sp.reshape(1, 1),
        W_ps.T, row(b_ps), aW_ps.T, ab_ps.reshape(1, 1),
        Wn1.T, row(bn1), Wn2.T, row(bn2),
        Wp1.T, row(bp1), Wp2.T, row(bp2))
    return (sites_new, bonds, sites_p_new, bonds_sp, bonds_ps)
